# Initial kernel scaffold; baseline (speedup 1.0000x reference)
import functools

import jax
import jax.numpy as jnp
from jax import lax
from jax.experimental import pallas as pl
from jax.experimental.pallas import tpu as pltpu
from jax.experimental.pallas import tpu_sc as plsc

T = 2048
D = 768
FF = 1024
E = 64
K = 2
TK = T * K
TM = 128
NT = TK // TM
NS = E + NT - 1

_SQRT_HALF = 0.7071067811865476


def _router_body(hs_ref, gw_ref, idx_ref, w_ref):
    x = hs_ref[...]
    logits = lax.dot_general(x, gw_ref[...], (((1,), (1,)), ((), ())),
                             preferred_element_type=jnp.float32)
    col = lax.broadcasted_iota(jnp.int32, (T, E), 1)
    m1 = jnp.max(logits, axis=1, keepdims=True)
    a1 = jnp.min(jnp.where(logits == m1, col, E), axis=1, keepdims=True)
    l2 = jnp.where(col == a1, -jnp.inf, logits)
    m2 = jnp.max(l2, axis=1, keepdims=True)
    a2 = jnp.min(jnp.where(l2 == m2, col, E), axis=1, keepdims=True)
    w1 = 1.0 / (1.0 + jnp.exp(m2 - m1))
    idx_ref[...] = jnp.concatenate([a1, a2], axis=1)
    w_ref[...] = jnp.concatenate([w1, 1.0 - w1], axis=1)


def _router(hs2d, gate_w):
    return pl.pallas_call(
        _router_body,
        out_shape=(jax.ShapeDtypeStruct((T, K), jnp.int32),
                   jax.ShapeDtypeStruct((T, K), jnp.float32)),
    )(hs2d, gate_w)


def _ffn_body(meta_ref, xs_ref, wg_ref, wu_ref, wd_ref, wr_ref, out_ref):
    s = pl.program_id(0)
    lo = meta_ref[2, s]
    hi = meta_ref[3, s]
    x = xs_ref[...]
    wg = wg_ref[0]
    wu = wu_ref[0]
    wd = wd_ref[0]
    h = lax.dot_general(x, wg, (((1,), (1,)), ((), ())),
                        preferred_element_type=jnp.float32)
    u = lax.dot_general(x, wu, (((1,), (1,)), ((), ())),
                        preferred_element_type=jnp.float32)
    g = 0.5 * h * (1.0 + lax.erf(h * _SQRT_HALF))
    y = lax.dot_general(g * u, wd, (((1,), (1,)), ((), ())),
                        preferred_element_type=jnp.float32)
    y = y * wr_ref[:, 0:1]
    row = lax.broadcasted_iota(jnp.int32, (TM, 1), 0)
    mask = (row >= lo) & (row < hi)
    out_ref[...] = jnp.where(mask, y, out_ref[...])


def _grouped_ffn(meta, xs, Wg, Wu, Wd, w_rep):
    grid_spec = pltpu.PrefetchScalarGridSpec(
        num_scalar_prefetch=1,
        grid=(NS,),
        in_specs=[
            pl.BlockSpec((TM, D), lambda s, m: (m[1, s], 0)),
            pl.BlockSpec((1, FF, D), lambda s, m: (m[0, s], 0, 0)),
            pl.BlockSpec((1, FF, D), lambda s, m: (m[0, s], 0, 0)),
            pl.BlockSpec((1, D, FF), lambda s, m: (m[0, s], 0, 0)),
            pl.BlockSpec((TM, 128), lambda s, m: (m[1, s], 0)),
        ],
        out_specs=pl.BlockSpec((TM, D), lambda s, m: (m[1, s], 0)),
    )
    return pl.pallas_call(
        _ffn_body,
        grid_spec=grid_spec,
        out_shape=jax.ShapeDtypeStruct((TK, D), jnp.float32),
        compiler_params=pltpu.CompilerParams(
            dimension_semantics=("arbitrary",)),
    )(meta, xs, Wg, Wu, Wd, w_rep)


_MESH = plsc.VectorSubcoreMesh(core_axis_name="c", subcore_axis_name="s")
_NW = 32
_BPW = TK // _NW
_CPW = T // _NW


@functools.partial(
    pl.kernel, mesh=_MESH,
    out_type=jax.ShapeDtypeStruct((TK, D), jnp.float32),
    scratch_types=[
        pltpu.VMEM((_BPW,), jnp.int32),
        pltpu.VMEM((_BPW, D), jnp.float32),
        pltpu.SemaphoreType.DMA,
    ],
)
def _sc_gather(hs_hbm, tok_hbm, out_hbm, idx_v, rows_v, sem):
    wid = lax.axis_index("s") * 2 + lax.axis_index("c")
    base = wid * _BPW
    pltpu.sync_copy(tok_hbm.at[pl.ds(base, _BPW)], idx_v)
    pltpu.async_copy(hs_hbm.at[idx_v], rows_v, sem).wait()
    pltpu.sync_copy(rows_v, out_hbm.at[pl.ds(base, _BPW)])


@functools.partial(
    pl.kernel, mesh=_MESH,
    out_type=jax.ShapeDtypeStruct((T, D), jnp.float32),
    scratch_types=[
        pltpu.VMEM((_CPW,), jnp.int32),
        pltpu.VMEM((_CPW,), jnp.int32),
        pltpu.VMEM((_CPW, D), jnp.float32),
        pltpu.VMEM((_CPW, D), jnp.float32),
        pltpu.SemaphoreType.DMA,
        pltpu.SemaphoreType.DMA,
    ],
)
def _sc_combine(ysw_hbm, pos0_hbm, pos1_hbm, out_hbm,
                i0_v, i1_v, r0_v, r1_v, s0, s1):
    wid = lax.axis_index("s") * 2 + lax.axis_index("c")
    base = wid * _CPW
    pltpu.sync_copy(pos0_hbm.at[pl.ds(base, _CPW)], i0_v)
    pltpu.sync_copy(pos1_hbm.at[pl.ds(base, _CPW)], i1_v)
    c0 = pltpu.async_copy(ysw_hbm.at[i0_v], r0_v, s0)
    c1 = pltpu.async_copy(ysw_hbm.at[i1_v], r1_v, s1)
    c0.wait()
    c1.wait()

    def row_add(j, carry):
        for c in range(D // 16):
            sl = pl.ds(c * 16, 16)
            r0_v[j, sl] = r0_v[j, sl] + r1_v[j, sl]
        return carry

    lax.fori_loop(0, _CPW, row_add, 0)
    pltpu.sync_copy(r0_v, out_hbm.at[pl.ds(base, _CPW)])


def _metadata(idx, wts):
    flat_e = idx.reshape(-1)
    flat_w = wts.reshape(-1)
    sort_idx = jnp.argsort(flat_e).astype(jnp.int32)
    sort_tok = (sort_idx // K).astype(jnp.int32)
    inv = jnp.zeros((TK,), jnp.int32).at[sort_idx].set(
        jnp.arange(TK, dtype=jnp.int32))
    pos = inv.reshape(T, K)
    w_sorted = flat_w[sort_idx]

    counts = jnp.bincount(flat_e, length=E).astype(jnp.int32)
    ends = jnp.cumsum(counts).astype(jnp.int32)
    starts = ends - counts
    first_tile = starts // TM
    last_tile = jnp.maximum(ends - 1, 0) // TM
    tiles_e = jnp.where(counts > 0, last_tile - first_tile + 1, 0)
    cum = jnp.cumsum(tiles_e)
    step_base = cum - tiles_e
    total = cum[-1]

    s_ids = jnp.minimum(jnp.arange(NS, dtype=jnp.int32), total - 1)
    e_of_s = jnp.searchsorted(cum, s_ids, side="right").astype(jnp.int32)
    e_of_s = jnp.clip(e_of_s, 0, E - 1)
    tile_of_s = first_tile[e_of_s] + (s_ids - step_base[e_of_s])
    t0 = tile_of_s * TM
    lo = jnp.clip(starts[e_of_s] - t0, 0, TM)
    hi = jnp.clip(ends[e_of_s] - t0, 0, TM)
    valid = jnp.arange(NS, dtype=jnp.int32) < total
    hi = jnp.where(valid, hi, lo)
    meta = jnp.stack([e_of_s, tile_of_s, lo, hi]).astype(jnp.int32)
    return sort_tok, w_sorted, pos, meta


def kernel(hidden_states, gate_w, Wg, Wu, Wd):
    b, s, d = hidden_states.shape
    hs2d = hidden_states.reshape(T, D)
    idx, wts = _router(hs2d, gate_w)
    sort_tok, w_sorted, pos, meta = _metadata(idx, wts)
    xs = _sc_gather(hs2d, sort_tok)
    w_rep = jnp.broadcast_to(w_sorted[:, None], (TK, 128))
    ysw = _grouped_ffn(meta, xs, Wg, Wu, Wd, w_rep)
    final = _sc_combine(ysw, pos[:, 0], pos[:, 1])
    return final.reshape(b, s, d)


# speedup vs baseline: 7.5186x; 7.5186x over previous
import functools

import jax
import jax.numpy as jnp
from jax import lax
from jax.experimental import pallas as pl
from jax.experimental.pallas import tpu as pltpu
from jax.experimental.pallas import tpu_sc as plsc

T = 2048
D = 768
FF = 1024
E = 64
K = 2
TK = T * K
TM = 128
NT = TK // TM
NS = E + NT - 1

_SQRT_HALF = 0.7071067811865476


def _router_body(hs_ref, gw_ref, idx_ref, w_ref):
    x = hs_ref[...]
    logits = lax.dot_general(x, gw_ref[...], (((1,), (1,)), ((), ())),
                             preferred_element_type=jnp.float32)
    col = lax.broadcasted_iota(jnp.int32, (T, E), 1)
    m1 = jnp.max(logits, axis=1, keepdims=True)
    a1 = jnp.min(jnp.where(logits == m1, col, E), axis=1, keepdims=True)
    l2 = jnp.where(col == a1, -jnp.inf, logits)
    m2 = jnp.max(l2, axis=1, keepdims=True)
    a2 = jnp.min(jnp.where(l2 == m2, col, E), axis=1, keepdims=True)
    w1 = 1.0 / (1.0 + jnp.exp(m2 - m1))
    idx_ref[...] = jnp.concatenate([a1, a2], axis=1)
    w_ref[...] = jnp.concatenate([w1, 1.0 - w1], axis=1)


def _router(hs2d, gate_w):
    return pl.pallas_call(
        _router_body,
        out_shape=(jax.ShapeDtypeStruct((T, K), jnp.int32),
                   jax.ShapeDtypeStruct((T, K), jnp.float32)),
    )(hs2d, gate_w)


def _ffn_body(meta_ref, xs_ref, wg_ref, wu_ref, wd_ref, wr_ref, out_ref):
    s = pl.program_id(0)
    lo = meta_ref[2, s]
    hi = meta_ref[3, s]
    x = xs_ref[...]
    wg = wg_ref[0]
    wu = wu_ref[0]
    wd = wd_ref[0]
    h = lax.dot_general(x, wg, (((1,), (1,)), ((), ())),
                        preferred_element_type=jnp.float32)
    u = lax.dot_general(x, wu, (((1,), (1,)), ((), ())),
                        preferred_element_type=jnp.float32)
    g = 0.5 * h * (1.0 + lax.erf(h * _SQRT_HALF))
    y = lax.dot_general(g * u, wd, (((1,), (1,)), ((), ())),
                        preferred_element_type=jnp.float32)
    y = y * wr_ref[:, 0:1]
    row = lax.broadcasted_iota(jnp.int32, (TM, 1), 0)
    mask = (row >= lo) & (row < hi)
    out_ref[...] = jnp.where(mask, y, out_ref[...])


def _grouped_ffn(meta, xs, Wg, Wu, Wd, w_rep):
    grid_spec = pltpu.PrefetchScalarGridSpec(
        num_scalar_prefetch=1,
        grid=(NS,),
        in_specs=[
            pl.BlockSpec((TM, D), lambda s, m: (m[1, s], 0)),
            pl.BlockSpec((1, FF, D), lambda s, m: (m[0, s], 0, 0)),
            pl.BlockSpec((1, FF, D), lambda s, m: (m[0, s], 0, 0)),
            pl.BlockSpec((1, D, FF), lambda s, m: (m[0, s], 0, 0)),
            pl.BlockSpec((TM, 128), lambda s, m: (m[1, s], 0)),
        ],
        out_specs=pl.BlockSpec((TM, D), lambda s, m: (m[1, s], 0)),
    )
    return pl.pallas_call(
        _ffn_body,
        grid_spec=grid_spec,
        out_shape=jax.ShapeDtypeStruct((TK, D), jnp.float32),
        compiler_params=pltpu.CompilerParams(
            dimension_semantics=("arbitrary",)),
    )(meta, xs, Wg, Wu, Wd, w_rep)


_NW = 32
_BPW = TK // _NW
_CPW = T // _NW


@functools.cache
def _sc_kernels():
    mesh = plsc.VectorSubcoreMesh(core_axis_name="c", subcore_axis_name="s")

    @functools.partial(
        pl.kernel, mesh=mesh,
        out_type=jax.ShapeDtypeStruct((TK, D), jnp.float32),
        scratch_types=[
            pltpu.VMEM((_BPW,), jnp.int32),
            pltpu.VMEM((_BPW, D), jnp.float32),
            pltpu.SemaphoreType.DMA,
        ],
    )
    def sc_gather(hs_hbm, tok_hbm, out_hbm, idx_v, rows_v, sem):
        wid = lax.axis_index("s") * 2 + lax.axis_index("c")
        base = wid * _BPW
        pltpu.sync_copy(tok_hbm.at[pl.ds(base, _BPW)], idx_v)
        pltpu.async_copy(hs_hbm.at[idx_v], rows_v, sem).wait()
        pltpu.sync_copy(rows_v, out_hbm.at[pl.ds(base, _BPW)])

    @functools.partial(
        pl.kernel, mesh=mesh,
        out_type=jax.ShapeDtypeStruct((T, D), jnp.float32),
        scratch_types=[
            pltpu.VMEM((_CPW,), jnp.int32),
            pltpu.VMEM((_CPW,), jnp.int32),
            pltpu.VMEM((_CPW, D), jnp.float32),
            pltpu.VMEM((_CPW, D), jnp.float32),
            pltpu.SemaphoreType.DMA,
            pltpu.SemaphoreType.DMA,
        ],
    )
    def sc_combine(ysw_hbm, pos0_hbm, pos1_hbm, out_hbm,
                   i0_v, i1_v, r0_v, r1_v, s0, s1):
        wid = lax.axis_index("s") * 2 + lax.axis_index("c")
        base = wid * _CPW
        pltpu.sync_copy(pos0_hbm.at[pl.ds(base, _CPW)], i0_v)
        pltpu.sync_copy(pos1_hbm.at[pl.ds(base, _CPW)], i1_v)
        c0 = pltpu.async_copy(ysw_hbm.at[i0_v], r0_v, s0)
        c1 = pltpu.async_copy(ysw_hbm.at[i1_v], r1_v, s1)
        c0.wait()
        c1.wait()

        def row_add(j, carry):
            for c in range(D // 16):
                sl = pl.ds(c * 16, 16)
                r0_v[j, sl] = r0_v[j, sl] + r1_v[j, sl]
            return carry

        lax.fori_loop(0, _CPW, row_add, 0)
        pltpu.sync_copy(r0_v, out_hbm.at[pl.ds(base, _CPW)])

    return sc_gather, sc_combine


def _sc_gather(hs2d, sort_tok):
    return _sc_kernels()[0](hs2d, sort_tok)


def _sc_combine(ysw, pos0, pos1):
    return _sc_kernels()[1](ysw, pos0, pos1)


def _metadata(idx, wts):
    flat_e = idx.reshape(-1)
    flat_w = wts.reshape(-1)
    sort_idx = jnp.argsort(flat_e).astype(jnp.int32)
    sort_tok = (sort_idx // K).astype(jnp.int32)
    inv = jnp.zeros((TK,), jnp.int32).at[sort_idx].set(
        jnp.arange(TK, dtype=jnp.int32))
    pos = inv.reshape(T, K)
    w_sorted = flat_w[sort_idx]

    counts = jnp.bincount(flat_e, length=E).astype(jnp.int32)
    ends = jnp.cumsum(counts).astype(jnp.int32)
    starts = ends - counts
    first_tile = starts // TM
    last_tile = jnp.maximum(ends - 1, 0) // TM
    tiles_e = jnp.where(counts > 0, last_tile - first_tile + 1, 0)
    cum = jnp.cumsum(tiles_e)
    step_base = cum - tiles_e
    total = cum[-1]

    s_ids = jnp.minimum(jnp.arange(NS, dtype=jnp.int32), total - 1)
    e_of_s = jnp.searchsorted(cum, s_ids, side="right").astype(jnp.int32)
    e_of_s = jnp.clip(e_of_s, 0, E - 1)
    tile_of_s = first_tile[e_of_s] + (s_ids - step_base[e_of_s])
    t0 = tile_of_s * TM
    lo = jnp.clip(starts[e_of_s] - t0, 0, TM)
    hi = jnp.clip(ends[e_of_s] - t0, 0, TM)
    valid = jnp.arange(NS, dtype=jnp.int32) < total
    hi = jnp.where(valid, hi, lo)
    meta = jnp.stack([e_of_s, tile_of_s, lo, hi]).astype(jnp.int32)
    return sort_tok, w_sorted, pos, meta


def kernel(hidden_states, gate_w, Wg, Wu, Wd):
    b, s, d = hidden_states.shape
    hs2d = hidden_states.reshape(T, D)
    idx, wts = _router(hs2d, gate_w)
    sort_tok, w_sorted, pos, meta = _metadata(idx, wts)
    xs = _sc_gather(hs2d, sort_tok)
    w_rep = jnp.broadcast_to(w_sorted[:, None], (TK, 128))
    ysw = _grouped_ffn(meta, xs, Wg, Wu, Wd, w_rep)
    final = _sc_combine(ysw, pos[:, 0], pos[:, 1])
    return final.reshape(b, s, d)
